# baseline (speedup 1.0000x reference)
import functools

import jax
import jax.numpy as jnp
from jax.experimental import pallas as pl
from jax.experimental.pallas import tpu as pltpu


def _lag_kernel(x_ref, w1t_ref, b1r_ref, w2t_ref, b2r_ref, o_ref,
                slab_ref, acc_ref, gate_ref, *, nh, nbpc, h_t, inv_hw):
    t = pl.program_id(1)
    b_local = t // nh
    h = t % nh
    is_acc = b_local < nbpc

    @pl.when(jnp.logical_and(is_acc, h == 0))
    def _init():
        acc_ref[...] = jnp.zeros_like(acc_ref)

    @pl.when(is_acc)
    def _accumulate():
        tile = x_ref[0]
        W, C = tile.shape[1:]
        acc_ref[...] += jnp.sum(
            tile.astype(jnp.float32).reshape(h_t * W, C), axis=0)[None]
        slab_ref[b_local % 2, pl.ds(h * h_t, h_t)] = tile

    @pl.when(jnp.logical_and(is_acc, h == nh - 1))
    def _gate():
        pooled = acc_ref[...] * inv_hw
        y1 = jnp.dot(pooled, w1t_ref[...],
                     preferred_element_type=jnp.float32) + b1r_ref[...]
        y1 = jnp.maximum(y1, 0.0)
        y2 = jnp.dot(y1, w2t_ref[...],
                     preferred_element_type=jnp.float32) + b2r_ref[...]
        gate_ref[b_local % 2] = jax.nn.sigmoid(y2)

    @pl.when(b_local >= 1)
    def _rescale():
        p = (b_local - 1) % 2
        g = gate_ref[p, 0].astype(o_ref.dtype)
        o_ref[0] = slab_ref[p, pl.ds(h * h_t, h_t)] * g[None, None, :]


@jax.jit
def _ca_fused(x, w1, b1, w2, b2):
    B, C, H, W = x.shape
    mid = w1.shape[0]
    xt = jnp.transpose(x, (0, 2, 3, 1))

    w1t = jnp.transpose(w1)
    w2t = jnp.transpose(w2)
    b1r = b1.reshape(1, mid)
    b2r = b2.reshape(1, C)
    inv_hw = 1.0 / float(H * W)

    ncore = 2 if B % 2 == 0 else 1
    nbpc = B // ncore
    nh = 4 if H % 4 == 0 else 1
    h_t = H // nh

    def x_map(core, t):
        b_local = t // nh
        h = t % nh
        b_in = core * nbpc + jnp.minimum(b_local, nbpc - 1)
        h_in = jnp.where(b_local < nbpc, h, nh - 1)
        return (b_in, h_in, 0, 0)

    def o_map(core, t):
        b_local = t // nh
        h = t % nh
        b_out = core * nbpc + jnp.maximum(b_local - 1, 0)
        h_out = jnp.where(b_local >= 1, h, 0)
        return (b_out, h_out, 0, 0)

    out = pl.pallas_call(
        functools.partial(_lag_kernel, nh=nh, nbpc=nbpc, h_t=h_t,
                          inv_hw=inv_hw),
        out_shape=jax.ShapeDtypeStruct((B, H, W, C), x.dtype),
        grid_spec=pltpu.PrefetchScalarGridSpec(
            num_scalar_prefetch=0,
            grid=(ncore, (nbpc + 1) * nh),
            in_specs=[
                pl.BlockSpec((1, h_t, W, C), x_map),
                pl.BlockSpec((C, mid), lambda c, t: (0, 0)),
                pl.BlockSpec((1, mid), lambda c, t: (0, 0)),
                pl.BlockSpec((mid, C), lambda c, t: (0, 0)),
                pl.BlockSpec((1, C), lambda c, t: (0, 0)),
            ],
            out_specs=pl.BlockSpec((1, h_t, W, C), o_map),
            scratch_shapes=[
                pltpu.VMEM((2, H, W, C), x.dtype),
                pltpu.VMEM((1, C), jnp.float32),
                pltpu.VMEM((2, 1, C), jnp.float32),
            ],
        ),
        compiler_params=pltpu.CompilerParams(
            dimension_semantics=("parallel", "arbitrary")),
    )(xt, w1t, b1r, w2t, b2r)

    return jnp.transpose(out, (0, 3, 1, 2))


def kernel(x, w1, b1, w2, b2):
    return _ca_fused(x, w1, b1, w2, b2)


# speedup vs baseline: 1.5943x; 1.5943x over previous
import functools

import jax
import jax.numpy as jnp
from jax.experimental import pallas as pl
from jax.experimental.pallas import tpu as pltpu


def _fused_se_kernel(x_ref, w1t_ref, b1r_ref, w2t_ref, b2r_ref, o_ref, *,
                     inv_hw):
    H, W, C = x_ref.shape[1:]
    x = x_ref[0].reshape(H * W, C)
    pooled = (jnp.sum(x.astype(jnp.float32), axis=0) * inv_hw)[None, :]

    y1 = jnp.dot(pooled, w1t_ref[...],
                 preferred_element_type=jnp.float32) + b1r_ref[...]
    y1 = jnp.maximum(y1, 0.0)
    y2 = jnp.dot(y1, w2t_ref[...],
                 preferred_element_type=jnp.float32) + b2r_ref[...]
    gate = jax.nn.sigmoid(y2).astype(o_ref.dtype)

    o_ref[...] = x_ref[...] * gate[0][None, None, None, :]


@jax.jit
def _ca_fused(x, w1, b1, w2, b2):
    B, C, H, W = x.shape
    mid = w1.shape[0]
    xt = jnp.transpose(x, (0, 2, 3, 1))

    w1t = jnp.transpose(w1)
    w2t = jnp.transpose(w2)
    b1r = b1.reshape(1, mid)
    b2r = b2.reshape(1, C)
    inv_hw = 1.0 / float(H * W)

    out = pl.pallas_call(
        functools.partial(_fused_se_kernel, inv_hw=inv_hw),
        out_shape=jax.ShapeDtypeStruct((B, H, W, C), x.dtype),
        grid=(B,),
        in_specs=[
            pl.BlockSpec((1, H, W, C), lambda b: (b, 0, 0, 0)),
            pl.BlockSpec((C, mid), lambda b: (0, 0)),
            pl.BlockSpec((1, mid), lambda b: (0, 0)),
            pl.BlockSpec((mid, C), lambda b: (0, 0)),
            pl.BlockSpec((1, C), lambda b: (0, 0)),
        ],
        out_specs=pl.BlockSpec((1, H, W, C), lambda b: (b, 0, 0, 0)),
        compiler_params=pltpu.CompilerParams(
            dimension_semantics=("parallel",)),
    )(xt, w1t, b1r, w2t, b2r)

    return jnp.transpose(out, (0, 3, 1, 2))


def kernel(x, w1, b1, w2, b2):
    return _ca_fused(x, w1, b1, w2, b2)


# speedup vs baseline: 1.5974x; 1.0019x over previous
import functools

import jax
import jax.numpy as jnp
from jax.experimental import pallas as pl
from jax.experimental.pallas import tpu as pltpu


def _fused_se_kernel(x_ref, w1_ref, b1r_ref, w2_ref, b2r_ref, o_ref, *,
                     inv_hw):
    H, W, C = x_ref.shape[1:]
    x = x_ref[0].reshape(H * W, C)
    pooled = (jnp.sum(x.astype(jnp.float32), axis=0) * inv_hw)[None, :]

    y1 = jnp.dot(pooled, jnp.transpose(w1_ref[...]),
                 preferred_element_type=jnp.float32) + b1r_ref[...]
    y1 = jnp.maximum(y1, 0.0)
    y2 = jnp.dot(y1, jnp.transpose(w2_ref[...]),
                 preferred_element_type=jnp.float32) + b2r_ref[...]
    gate = jax.nn.sigmoid(y2).astype(o_ref.dtype)

    o_ref[...] = x_ref[...] * gate[0][None, None, None, :]


@jax.jit
def _ca_fused(x, w1, b1, w2, b2):
    B, C, H, W = x.shape
    mid = w1.shape[0]
    xt = jnp.transpose(x, (0, 2, 3, 1))

    b1r = b1.reshape(1, mid)
    b2r = b2.reshape(1, C)
    inv_hw = 1.0 / float(H * W)

    out = pl.pallas_call(
        functools.partial(_fused_se_kernel, inv_hw=inv_hw),
        out_shape=jax.ShapeDtypeStruct((B, H, W, C), x.dtype),
        grid=(B,),
        in_specs=[
            pl.BlockSpec((1, H, W, C), lambda b: (b, 0, 0, 0)),
            pl.BlockSpec((mid, C), lambda b: (0, 0)),
            pl.BlockSpec((1, mid), lambda b: (0, 0)),
            pl.BlockSpec((C, mid), lambda b: (0, 0)),
            pl.BlockSpec((1, C), lambda b: (0, 0)),
        ],
        out_specs=pl.BlockSpec((1, H, W, C), lambda b: (b, 0, 0, 0)),
        compiler_params=pltpu.CompilerParams(
            dimension_semantics=("parallel",)),
    )(xt, w1, b1r, w2, b2r)

    return jnp.transpose(out, (0, 3, 1, 2))


def kernel(x, w1, b1, w2, b2):
    return _ca_fused(x, w1, b1, w2, b2)
